# baseline (speedup 1.0000x reference)
import functools

import jax
import jax.numpy as jnp
from jax import lax
from jax.experimental import pallas as pl
from jax.experimental.pallas import tpu as pltpu
from jax.experimental.pallas import tpu_sc as plsc

BLOCK_S = 1024


def _add_kernel(x_ref, pos_ref, o_ref):
    o_ref[...] = x_ref[...] + pos_ref[...]


def _kernel_tc(x, pos_table):
    batch, seq_len, embed_dim = x.shape
    pos = pos_table[:seq_len]
    num_s = seq_len // BLOCK_S
    return pl.pallas_call(
        _add_kernel,
        grid=(num_s, batch),
        in_specs=[
            pl.BlockSpec((1, BLOCK_S, embed_dim), lambda i, j: (j, i, 0)),
            pl.BlockSpec((BLOCK_S, embed_dim), lambda i, j: (i, 0)),
        ],
        out_specs=pl.BlockSpec((1, BLOCK_S, embed_dim), lambda i, j: (j, i, 0)),
        out_shape=jax.ShapeDtypeStruct(x.shape, x.dtype),
    )(x, pos)


_CHUNK_ROWS = 16


def _kernel_sc(x, pos_table):
    batch, seq_len, embed_dim = x.shape
    info = plsc.get_sparse_core_info()
    nc, ns = info.num_cores, info.num_subcores
    nw = nc * ns
    tiles_per_batch = nw // batch
    rows_per_tile = seq_len // tiles_per_batch
    nchunks = rows_per_tile // _CHUNK_ROWS
    chunk_elems = _CHUNK_ROWS * embed_dim

    mesh = plsc.VectorSubcoreMesh(core_axis_name="c", subcore_axis_name="s")

    @functools.partial(
        pl.kernel,
        out_type=jax.ShapeDtypeStruct((batch * seq_len * embed_dim,), x.dtype),
        mesh=mesh,
        scratch_types=[
            pltpu.VMEM((chunk_elems,), jnp.float32),
            pltpu.VMEM((chunk_elems,), jnp.float32),
        ],
    )
    def k(x_hbm, pos_hbm, out_hbm, vx, vp):
        wid = lax.axis_index("s") * nc + lax.axis_index("c")
        b = wid // tiles_per_batch
        j = wid % tiles_per_batch

        def chunk_body(c, _):
            srow = j * rows_per_tile + c * _CHUNK_ROWS
            xoff = (b * seq_len + srow) * embed_dim
            poff = srow * embed_dim
            pltpu.sync_copy(x_hbm.at[pl.ds(xoff, chunk_elems)], vx)
            pltpu.sync_copy(pos_hbm.at[pl.ds(poff, chunk_elems)], vp)

            def add_body(i, _):
                sl = pl.ds(pl.multiple_of(i * 16, 16), 16)
                vx[sl] = vx[sl] + vp[sl]
                return 0

            lax.fori_loop(0, chunk_elems // 16, add_body, 0)
            pltpu.sync_copy(vx, out_hbm.at[pl.ds(xoff, chunk_elems)])
            return 0

        lax.fori_loop(0, nchunks, chunk_body, 0)

    out = k(x.reshape(-1), pos_table[:seq_len].reshape(-1))
    return out.reshape(x.shape)


def kernel(x, pos_table):
    return _kernel_sc(x, pos_table)


# speedup vs baseline: 8.0920x; 8.0920x over previous
import functools

import jax
import jax.numpy as jnp
from jax import lax
from jax.experimental import pallas as pl
from jax.experimental.pallas import tpu as pltpu
from jax.experimental.pallas import tpu_sc as plsc

BLOCK_S = 1024


def _add_kernel(x_ref, pos_ref, o_ref):
    o_ref[...] = x_ref[...] + pos_ref[...]


def _kernel_tc(x, pos_table):
    batch, seq_len, embed_dim = x.shape
    pos = pos_table[:seq_len]
    num_s = seq_len // BLOCK_S
    return pl.pallas_call(
        _add_kernel,
        grid=(num_s, batch),
        in_specs=[
            pl.BlockSpec((1, BLOCK_S, embed_dim), lambda i, j: (j, i, 0)),
            pl.BlockSpec((BLOCK_S, embed_dim), lambda i, j: (i, 0)),
        ],
        out_specs=pl.BlockSpec((1, BLOCK_S, embed_dim), lambda i, j: (j, i, 0)),
        out_shape=jax.ShapeDtypeStruct(x.shape, x.dtype),
    )(x, pos)


_CHUNK_ROWS = 16


def _kernel_sc(x, pos_table):
    batch, seq_len, embed_dim = x.shape
    info = plsc.get_sparse_core_info()
    nc, ns = info.num_cores, info.num_subcores
    nw = nc * ns
    tiles_per_batch = nw // batch
    rows_per_tile = seq_len // tiles_per_batch
    nchunks = rows_per_tile // _CHUNK_ROWS
    chunk_elems = _CHUNK_ROWS * embed_dim

    mesh = plsc.VectorSubcoreMesh(core_axis_name="c", subcore_axis_name="s")

    @functools.partial(
        pl.kernel,
        out_type=jax.ShapeDtypeStruct((batch * seq_len * embed_dim,), x.dtype),
        mesh=mesh,
        scratch_types=[
            pltpu.VMEM((chunk_elems,), jnp.float32),
            pltpu.VMEM((chunk_elems,), jnp.float32),
        ],
    )
    def k(x_hbm, pos_hbm, out_hbm, vx, vp):
        wid = lax.axis_index("s") * nc + lax.axis_index("c")
        b = wid // tiles_per_batch
        j = wid % tiles_per_batch

        def chunk_body(c, _):
            srow = j * rows_per_tile + c * _CHUNK_ROWS
            xoff = (b * seq_len + srow) * embed_dim
            poff = srow * embed_dim
            pltpu.sync_copy(x_hbm.at[pl.ds(xoff, chunk_elems)], vx)
            pltpu.sync_copy(pos_hbm.at[pl.ds(poff, chunk_elems)], vp)

            def add_body(i, _):
                sl = pl.ds(pl.multiple_of(i * 16, 16), 16)
                vx[sl] = vx[sl] + vp[sl]
                return 0

            lax.fori_loop(0, chunk_elems // 16, add_body, 0)
            pltpu.sync_copy(vx, out_hbm.at[pl.ds(xoff, chunk_elems)])
            return 0

        lax.fori_loop(0, nchunks, chunk_body, 0)

    out = k(x.reshape(-1), pos_table[:seq_len].reshape(-1))
    return out.reshape(x.shape)


def kernel(x, pos_table):
    return _kernel_tc(x, pos_table)
